# baseline (speedup 1.0000x reference)
import jax
import jax.numpy as jnp
from jax.experimental import pallas as pl
from jax.experimental.pallas import tpu as pltpu
from jax.experimental.pallas import tpu_sc as plsc

_WINDOW = 256


def kernel(model_input, table):
    batch, seq = model_input.shape
    num_idx = batch * seq
    rows, dim = table.shape
    indices = model_input.reshape(num_idx)

    padded = jnp.pad(table, ((0, 0), (0, 128 - dim)))

    mesh = plsc.VectorSubcoreMesh(core_axis_name="core",
                                  subcore_axis_name="subcore")

    @pl.kernel(
        out_type=jax.ShapeDtypeStruct((num_idx, 128), table.dtype),
        mesh=mesh,
    )
    def gather(tab_hbm, idx_hbm, out_hbm):
        def body(idx_vmem, out_vmem):
            pltpu.sync_copy(tab_hbm.at[idx_vmem], out_vmem)

        pltpu.emit_pipeline(
            body,
            grid=(num_idx // _WINDOW,),
            in_specs=[pl.BlockSpec((_WINDOW,),
                                   index_map=lambda i: (i,))],
            out_specs=[pl.BlockSpec((_WINDOW, 128),
                                    index_map=lambda i: (i, 0))],
            core_axis_name=("core", "subcore"),
            dimension_semantics=(pltpu.PARALLEL,),
        )(idx_hbm, out_hbm)

    out = gather(padded, indices)
    return out[:, :dim].reshape(batch, seq, dim)
